# Initial kernel scaffold; baseline (speedup 1.0000x reference)
import jax
import jax.numpy as jnp
from jax.experimental import pallas as pl


def kernel(x, edge_index, batch, edge_attr, W_r, relation_embedding, bn0_w, bn0_b, bn1_w, bn1_b, Wl0, bl0, Wl1, bl1, bnh_w, bnh_b, pW1, pb1, pW2, pb2):
    raise NotImplementedError("write your pallas kernel here")


# speedup vs baseline: 1.1789x; 1.1789x over previous
import functools

import jax
import jax.numpy as jnp
from jax import lax
from jax.experimental import pallas as pl
from jax.experimental.pallas import tpu as pltpu

N = 10000
E = 160000
D = 128
R = 16
G = 256
H = 3 * D


def _bn_body(x_ref, w_ref, b_ref, o_ref):
    x = x_ref[...]
    mu = jnp.mean(x, axis=0, keepdims=True)
    var = jnp.mean(jnp.square(x - mu), axis=0, keepdims=True)
    o_ref[...] = (x - mu) * jax.lax.rsqrt(var + 1e-5) * w_ref[...] + b_ref[...]


def _batchnorm(x, w, b):
    n, d = x.shape
    return pl.pallas_call(
        _bn_body,
        out_shape=jax.ShapeDtypeStruct((n, d), jnp.float32),
    )(x, w.reshape(1, d), b.reshape(1, d))


def _transform_body(h_ref, w_ref, o_ref):
    o_ref[0, ...] = jax.lax.dot_general(
        h_ref[...], w_ref[0, ...], (((1,), (1,)), ((), ())),
        preferred_element_type=jnp.float32, precision=lax.Precision.HIGHEST)


def _transform(h, W_r):
    NB = 10
    nb = N // NB
    return pl.pallas_call(
        _transform_body,
        grid=(R, NB),
        in_specs=[
            pl.BlockSpec((nb, D), lambda r, i: (i, 0)),
            pl.BlockSpec((1, D, D), lambda r, i: (r, 0, 0)),
        ],
        out_specs=pl.BlockSpec((1, nb, D), lambda r, i: (r, i, 0)),
        out_shape=jax.ShapeDtypeStruct((R, N, D), jnp.float32),
    )(h, W_r)


def _score_body(hs_ref, hd_ref, rel_ref, emb_ref, s_ref, m_ref):
    rel = rel_ref[0, ...]
    onehot = (rel[:, None] == lax.broadcasted_iota(jnp.int32, (1, R), 1)
              ).astype(jnp.float32)
    e_r = jax.lax.dot_general(onehot, emb_ref[...], (((1,), (0,)), ((), ())),
                              preferred_element_type=jnp.float32, precision=lax.Precision.HIGHEST)
    t = jnp.tanh(hs_ref[...] + e_r)
    s = jnp.sum(hd_ref[...] * t, axis=1)
    s_ref[0, ...] = s
    bm = jnp.max(s)
    i = pl.program_id(0)

    @pl.when(i == 0)
    def _init():
        m_ref[0, 0] = bm

    @pl.when(i > 0)
    def _acc():
        m_ref[0, 0] = jnp.maximum(m_ref[0, 0], bm)


def _scores(h_src, h_dst, rel, rel_emb):
    EB = 25
    eb = E // EB
    s, m = pl.pallas_call(
        _score_body,
        grid=(EB,),
        in_specs=[
            pl.BlockSpec((eb, D), lambda i: (i, 0)),
            pl.BlockSpec((eb, D), lambda i: (i, 0)),
            pl.BlockSpec((1, eb), lambda i: (0, i)),
            pl.BlockSpec((R, D), lambda i: (0, 0)),
        ],
        out_specs=[
            pl.BlockSpec((1, eb), lambda i: (0, i)),
            pl.BlockSpec((1, 1), lambda i: (0, 0), memory_space=pltpu.SMEM),
        ],
        out_shape=[
            jax.ShapeDtypeStruct((1, E), jnp.float32),
            jax.ShapeDtypeStruct((1, 1), jnp.float32),
        ],
    )(h_src, h_dst, rel.reshape(1, E), rel_emb)
    return s.reshape(E), m[0, 0]


def _linear_body(s_ref, d_ref, w_ref, b_ref, o_ref):
    agg = s_ref[...] / jnp.maximum(d_ref[...], 1e-30)
    y = jax.lax.dot_general(agg, w_ref[...], (((1,), (1,)), ((), ())),
                            preferred_element_type=jnp.float32, precision=lax.Precision.HIGHEST)
    o_ref[...] = jax.nn.relu(y + b_ref[...])


def _linear_relu(S, denom, Wl, bl):
    NB = 10
    nb = N // NB
    denom = denom.reshape(N, 1)
    return pl.pallas_call(
        _linear_body,
        grid=(NB,),
        in_specs=[
            pl.BlockSpec((nb, D), lambda i: (i, 0)),
            pl.BlockSpec((nb, 1), lambda i: (i, 0)),
            pl.BlockSpec((D, D), lambda i: (0, 0)),
            pl.BlockSpec((1, D), lambda i: (0, 0)),
        ],
        out_specs=pl.BlockSpec((nb, D), lambda i: (i, 0)),
        out_shape=jax.ShapeDtypeStruct((N, D), jnp.float32),
    )(S, denom, Wl, bl.reshape(1, D))


def _head_body(cat_ref, batch_ref, w_ref, b_ref, pw1_ref, pb1_ref,
               pw2_ref, pb2_ref, o_ref):
    onehot = (batch_ref[0, :][:, None] ==
              lax.broadcasted_iota(jnp.int32, (1, G), 1)).astype(jnp.float32)
    sums = jax.lax.dot_general(onehot, cat_ref[...], (((0,), (0,)), ((), ())),
                               preferred_element_type=jnp.float32, precision=lax.Precision.HIGHEST)
    counts = jnp.sum(onehot, axis=0)
    pooled = sums / jnp.maximum(counts, 1.0)[:, None]
    mu = jnp.mean(pooled, axis=0, keepdims=True)
    var = jnp.mean(jnp.square(pooled - mu), axis=0, keepdims=True)
    pooled = (pooled - mu) * jax.lax.rsqrt(var + 1e-5) * w_ref[...] + b_ref[...]
    y = jax.nn.relu(
        jax.lax.dot_general(pooled, pw1_ref[...], (((1,), (1,)), ((), ())),
                            preferred_element_type=jnp.float32, precision=lax.Precision.HIGHEST) + pb1_ref[...])
    o_ref[...] = jax.lax.dot_general(
        y, pw2_ref[...], (((1,), (1,)), ((), ())),
        preferred_element_type=jnp.float32, precision=lax.Precision.HIGHEST) + pb2_ref[...]


def _pool_head(cat, batch, bnh_w, bnh_b, pW1, pb1, pW2, pb2):
    return pl.pallas_call(
        _head_body,
        out_shape=jax.ShapeDtypeStruct((G, H), jnp.float32),
    )(cat, batch.reshape(1, N), bnh_w.reshape(1, H), bnh_b.reshape(1, H),
      pW1, pb1.reshape(1, H), pW2, pb2.reshape(1, H))


def _gather_rows(table, idx):
    return table[idx]


def _aggregate(scores, gmax, src, dst, h):
    smax = jax.ops.segment_max(scores, dst, num_segments=N)
    sexp = jnp.exp(scores - smax[dst])
    denom = jax.ops.segment_sum(sexp, dst, num_segments=N)
    S = jax.ops.segment_sum(sexp[:, None] * h[src], dst, num_segments=N)
    return S, denom


def _cagat_layer(h, src, dst, rel, idx_src, idx_dst, W_r, rel_emb, Wl, bl):
    xt = _transform(h, W_r).reshape(R * N, D)
    h_src = _gather_rows(xt, idx_src)
    h_dst = _gather_rows(xt, idx_dst)
    scores, gmax = _scores(h_src, h_dst, rel, rel_emb)
    S, denom = _aggregate(scores, gmax, src, dst, h)
    return _linear_relu(S, denom, Wl, bl)


def kernel(x, edge_index, batch, edge_attr, W_r, relation_embedding,
           bn0_w, bn0_b, bn1_w, bn1_b, Wl0, bl0, Wl1, bl1,
           bnh_w, bnh_b, pW1, pb1, pW2, pb2):
    src = edge_index[0]
    dst = edge_index[1]
    rel = edge_attr
    idx_src = rel * N + src
    idx_dst = rel * N + dst

    h0 = _batchnorm(x, bn0_w, bn0_b)
    h1 = _cagat_layer(h0, src, dst, rel, idx_src, idx_dst,
                      W_r, relation_embedding, Wl0, bl0)
    h2in = _batchnorm(h1, bn1_w, bn1_b)
    h2 = _cagat_layer(h2in, src, dst, rel, idx_src, idx_dst,
                      W_r, relation_embedding, Wl1, bl1)

    cat = jnp.concatenate([x, h1, h2], axis=1)
    return _pool_head(cat, batch, bnh_w, bnh_b, pW1, pb1, pW2, pb2)
